# Initial kernel scaffold; baseline (speedup 1.0000x reference)
import jax
import jax.numpy as jnp
from jax.experimental import pallas as pl


def kernel(h, e, edge_index, params):
    raise NotImplementedError("write your pallas kernel here")


# speedup vs baseline: 11.7883x; 11.7883x over previous
import functools

import jax
import jax.numpy as jnp
from jax import lax
from jax.experimental import pallas as pl
from jax.experimental.pallas import tpu as pltpu
from jax.experimental.pallas import tpu_sc as plsc

N_NODES = 10000
N_EDGES = 160000
NW = 32
NCHUNK = 40
CH = 128
E_PAD = NW * NCHUNK * CH
N_PAD = 10240
ROWS_PER_SUB = N_PAD // 16

_f32 = jnp.float32


def _mm_body(x_ref, w_ref, b_ref, o_ref, *, act):
    y = jnp.dot(x_ref[...], w_ref[...], preferred_element_type=_f32) + b_ref[...]
    if act == 'relu':
        y = jnp.maximum(y, 0.0)
    o_ref[...] = y


def _mm(x, w, b, act=None, blk=1024):
    m, k = x.shape
    n = w.shape[1]
    assert m % blk == 0
    return pl.pallas_call(
        functools.partial(_mm_body, act=act),
        grid=(m // blk,),
        in_specs=[
            pl.BlockSpec((blk, k), lambda i: (i, 0)),
            pl.BlockSpec((k, n), lambda i: (0, 0)),
            pl.BlockSpec((1, n), lambda i: (0, 0)),
        ],
        out_specs=pl.BlockSpec((blk, n), lambda i: (i, 0)),
        out_shape=jax.ShapeDtypeStruct((m, n), _f32),
    )(x, w, b.reshape(1, n))


def _rowmax_body(x_ref, o_ref):
    i = pl.program_id(0)
    bm = jnp.max(x_ref[...], axis=0, keepdims=True)

    @pl.when(i == 0)
    def _():
        o_ref[...] = bm

    @pl.when(i > 0)
    def _():
        o_ref[...] = jnp.maximum(o_ref[...], bm)


def _rowmax(x, blk=1024):
    m, n = x.shape
    return pl.pallas_call(
        _rowmax_body,
        grid=(m // blk,),
        in_specs=[pl.BlockSpec((blk, n), lambda i: (i, 0))],
        out_specs=pl.BlockSpec((1, n), lambda i: (0, 0)),
        out_shape=jax.ShapeDtypeStruct((1, n), _f32),
    )(x)


def _elu(x):
    return jnp.where(x > 0, x, jnp.exp(jnp.minimum(x, 0.0)) - 1.0)


def _logits_body(ep_ref, g1_ref, g2_ref, ein_ref, a2_ref, a3_ref, w2_ref,
                 w3_ref, lr_ref, enew_ref, *, H):
    ep_blk = ep_ref[...]
    g1 = g1_ref[...]
    g2 = g2_ref[...]
    eA = ep_blk[:, :128]
    epart = ep_blk[:, 128:128 + H]
    logit = (epart
             + jnp.dot(g1, a2_ref[...], preferred_element_type=_f32)
             + jnp.dot(g2, a3_ref[...], preferred_element_type=_f32))
    lr_ref[...] = jnp.where(logit > 0, logit, 0.01 * logit)
    enew_ref[...] = _elu(
        eA
        + jnp.dot(g1, w2_ref[...], preferred_element_type=_f32)
        + jnp.dot(g2, w3_ref[...], preferred_element_type=_f32)
    ) + ein_ref[...]


def _edge_logits(EP, G1, G2, e_in, A2M, A3M, W2bd, W3bd, H, blk=1024):
    m = EP.shape[0]
    return pl.pallas_call(
        functools.partial(_logits_body, H=H),
        grid=(m // blk,),
        in_specs=[
            pl.BlockSpec((blk, 144), lambda i: (i, 0)),
            pl.BlockSpec((blk, 128), lambda i: (i, 0)),
            pl.BlockSpec((blk, 128), lambda i: (i, 0)),
            pl.BlockSpec((blk, 128), lambda i: (i, 0)),
            pl.BlockSpec((128, H), lambda i: (0, 0)),
            pl.BlockSpec((128, H), lambda i: (0, 0)),
            pl.BlockSpec((128, 128), lambda i: (0, 0)),
            pl.BlockSpec((128, 128), lambda i: (0, 0)),
        ],
        out_specs=[
            pl.BlockSpec((blk, H), lambda i: (i, 0)),
            pl.BlockSpec((blk, 128), lambda i: (i, 0)),
        ],
        out_shape=[
            jax.ShapeDtypeStruct((m, H), _f32),
            jax.ShapeDtypeStruct((m, 128), _f32),
        ],
    )(EP, G1, G2, e_in, A2M, A3M, W2bd, W3bd)


def _weights_body(g1_ref, lr_ref, m_ref, expm_ref, padm_ref, wn_ref, wx_ref,
                  *, H, blk):
    i = pl.program_id(0)
    ex = jnp.exp(lr_ref[...] - m_ref[...])
    rows = i * blk + lax.broadcasted_iota(jnp.int32, (blk, H), 0)
    ex = jnp.where(rows < N_EDGES, ex, 0.0)
    zh = g1_ref[...]
    if H == 1:
        wn_ref[...] = ex * zh
        wx_ref[...] = ex * padm_ref[...]
    else:
        wn_ref[...] = jnp.dot(ex, expm_ref[...],
                              preferred_element_type=_f32) * zh
        wx_ref[...] = jnp.dot(ex, padm_ref[...], preferred_element_type=_f32)


def _edge_weights(G1, LR, M, EXPM, PADM, H, blk=1024):
    m = G1.shape[0]
    return pl.pallas_call(
        functools.partial(_weights_body, H=H, blk=blk),
        grid=(m // blk,),
        in_specs=[
            pl.BlockSpec((blk, 128), lambda i: (i, 0)),
            pl.BlockSpec((blk, H), lambda i: (i, 0)),
            pl.BlockSpec((1, H), lambda i: (0, 0)),
            pl.BlockSpec((H, 128), lambda i: (0, 0)),
            pl.BlockSpec((H, 128), lambda i: (0, 0)),
        ],
        out_specs=[
            pl.BlockSpec((blk, 128), lambda i: (i, 0)),
            pl.BlockSpec((blk, 128), lambda i: (i, 0)),
        ],
        out_shape=[
            jax.ShapeDtypeStruct((m, 128), _f32),
            jax.ShapeDtypeStruct((m, 128), _f32),
        ],
    )(G1, LR, M, EXPM, PADM)


def _finalize_body(n0_ref, n1_ref, s0_ref, s1_ref, hin_ref, expm_ref, o_ref,
                   *, H):
    num = n0_ref[...] + n1_ref[...]
    s = (s0_ref[...] + s1_ref[...])[:, :H]
    if H == 1:
        sex = s
    else:
        sex = jnp.dot(s, expm_ref[...], preferred_element_type=_f32)
    hagg = num / (sex + 1e-16)
    o_ref[...] = _elu(hagg) + hin_ref[...]


def _node_finalize(N0, N1, S0, S1, h_in, EXPM, H, blk=1000):
    m = N0.shape[0]
    return pl.pallas_call(
        functools.partial(_finalize_body, H=H),
        grid=(m // blk,),
        in_specs=[
            pl.BlockSpec((blk, 128), lambda i: (i, 0)),
            pl.BlockSpec((blk, 128), lambda i: (i, 0)),
            pl.BlockSpec((blk, 128), lambda i: (i, 0)),
            pl.BlockSpec((blk, 128), lambda i: (i, 0)),
            pl.BlockSpec((blk, 128), lambda i: (i, 0)),
            pl.BlockSpec((H, 128), lambda i: (0, 0)),
        ],
        out_specs=pl.BlockSpec((blk, 128), lambda i: (i, 0)),
        out_shape=jax.ShapeDtypeStruct((m, 128), _f32),
    )(N0, N1, S0, S1, h_in, EXPM)


def _readout_body(hs_ref, hd_ref, e_ref, wa_ref, wb_ref, wc_ref, b0_ref,
                  w1_ref, b1_ref, w2_ref, b2_ref, o_ref):
    y = (jnp.dot(hs_ref[...], wa_ref[...], preferred_element_type=_f32)
         + jnp.dot(hd_ref[...], wb_ref[...], preferred_element_type=_f32)
         + jnp.dot(e_ref[...], wc_ref[...], preferred_element_type=_f32)
         + b0_ref[...])
    y = jnp.maximum(y, 0.0)
    y = jnp.maximum(jnp.dot(y, w1_ref[...], preferred_element_type=_f32)
                    + b1_ref[...], 0.0)
    o_ref[...] = jnp.dot(y, w2_ref[...], preferred_element_type=_f32) + b2_ref[...]


def _readout(HS, HD, E, Wa, Wb, Wc, b0, W1, b1, W2p, b2p, blk=1024):
    m = HS.shape[0]
    d1 = Wa.shape[1]
    d2 = W1.shape[1]
    return pl.pallas_call(
        _readout_body,
        grid=(m // blk,),
        in_specs=[
            pl.BlockSpec((blk, 128), lambda i: (i, 0)),
            pl.BlockSpec((blk, 128), lambda i: (i, 0)),
            pl.BlockSpec((blk, 128), lambda i: (i, 0)),
            pl.BlockSpec((128, d1), lambda i: (0, 0)),
            pl.BlockSpec((128, d1), lambda i: (0, 0)),
            pl.BlockSpec((128, d1), lambda i: (0, 0)),
            pl.BlockSpec((1, d1), lambda i: (0, 0)),
            pl.BlockSpec((d1, d2), lambda i: (0, 0)),
            pl.BlockSpec((1, d2), lambda i: (0, 0)),
            pl.BlockSpec((d2, 128), lambda i: (0, 0)),
            pl.BlockSpec((1, 128), lambda i: (0, 0)),
        ],
        out_specs=pl.BlockSpec((blk, 128), lambda i: (i, 0)),
        out_shape=jax.ShapeDtypeStruct((m, 128), _f32),
    )(HS, HD, E, Wa, Wb, Wc, b0.reshape(1, d1), W1, b1.reshape(1, d2), W2p,
      b2p.reshape(1, 128))


@functools.lru_cache(maxsize=None)
def _make_gather2(n_rows, d1, d2):
    mesh = plsc.VectorSubcoreMesh(core_axis_name="c", subcore_axis_name="s")

    @functools.partial(
        pl.kernel, mesh=mesh,
        out_type=[
            jax.ShapeDtypeStruct((E_PAD, d1), _f32),
            jax.ShapeDtypeStruct((E_PAD, d2), _f32),
        ],
        scratch_types=[
            pltpu.VMEM((CH,), jnp.int32),
            pltpu.VMEM((CH,), jnp.int32),
            pltpu.VMEM((CH, d1), _f32),
            pltpu.VMEM((CH, d2), _f32),
            pltpu.SemaphoreType.DMA,
        ],
    )
    def k(t1, t2, i1, i2, o1, o2, iv1, iv2, b1, b2, sem):
        wid = lax.axis_index("s") * 2 + lax.axis_index("c")

        def body(c, carry):
            base = wid * (NCHUNK * CH) + c * CH
            pltpu.sync_copy(i1.at[wid, c], iv1)
            pltpu.sync_copy(i2.at[wid, c], iv2)
            pltpu.async_copy(t1.at[iv1], b1, sem).wait()
            pltpu.async_copy(t2.at[iv2], b2, sem).wait()
            pltpu.sync_copy(b1, o1.at[pl.ds(base, CH)])
            pltpu.sync_copy(b2, o2.at[pl.ds(base, CH)])
            return carry

        lax.fori_loop(0, NCHUNK, body, 0)

    return k


@functools.lru_cache(maxsize=None)
def _make_scatter(n_rows, d):
    mesh = plsc.VectorSubcoreMesh(core_axis_name="c", subcore_axis_name="s")

    @functools.partial(
        pl.kernel, mesh=mesh,
        out_type=jax.ShapeDtypeStruct((2, n_rows, d), _f32),
        scratch_types=[
            pltpu.VMEM((CH,), jnp.int32),
            pltpu.VMEM((CH, d), _f32),
            pltpu.VMEM_SHARED((n_rows, d), _f32),
            pltpu.SemaphoreType.DMA,
        ],
    )
    def k(w, idx, zer, acc, iv, wb, sh, sem):
        sid = lax.axis_index("s")
        cid = lax.axis_index("c")
        wid = sid * 2 + cid
        pltpu.sync_copy(zer.at[pl.ds(sid * ROWS_PER_SUB, ROWS_PER_SUB)],
                        sh.at[pl.ds(sid * ROWS_PER_SUB, ROWS_PER_SUB)])
        plsc.subcore_barrier()

        def body(c, carry):
            base = wid * (NCHUNK * CH) + c * CH
            pltpu.sync_copy(idx.at[wid, c], iv)
            pltpu.sync_copy(w.at[pl.ds(base, CH)], wb)
            pltpu.sync_copy(wb, sh.at[iv], add=True)
            return carry

        lax.fori_loop(0, NCHUNK, body, 0)
        plsc.subcore_barrier()
        pltpu.sync_copy(sh.at[pl.ds(sid * ROWS_PER_SUB, ROWS_PER_SUB)],
                        acc.at[cid, pl.ds(sid * ROWS_PER_SUB, ROWS_PER_SUB)])

    return k


def _block_diag(mats):
    rows = sum(m.shape[0] for m in mats)
    cols = sum(m.shape[1] for m in mats)
    out = jnp.zeros((rows, cols), _f32)
    r = c = 0
    for m in mats:
        out = out.at[r:r + m.shape[0], c:c + m.shape[1]].set(m)
        r += m.shape[0]
        c += m.shape[1]
    return out


def _prep_layer(heads):
    H = len(heads)
    od = heads[0]['fc_h'].shape[1]
    FCH = jnp.concatenate([p['fc_h'] for p in heads], 1)
    EW1 = jnp.concatenate([p['fc_e'] @ p['proj_W'][:od] for p in heads], 1)
    EA1 = jnp.stack([p['fc_e'] @ p['attn'][:od, 0] for p in heads], 1)
    pb = jnp.concatenate([p['proj_b'] for p in heads])
    pad_h = jnp.zeros((EW1.shape[0], 16 - H), _f32)
    W_e = jnp.concatenate([EW1, EA1, pad_h], 1)
    b_e = jnp.concatenate([pb, jnp.zeros((16,), _f32)])
    W2bd = _block_diag([p['proj_W'][od:2 * od] for p in heads])
    W3bd = _block_diag([p['proj_W'][2 * od:] for p in heads])
    A2M = _block_diag([p['attn'][od:2 * od] for p in heads])
    A3M = _block_diag([p['attn'][2 * od:] for p in heads])
    if H == 1:
        EXPM = jnp.ones((1, 128), _f32)
    else:
        EXPM = jnp.kron(jnp.eye(H, dtype=_f32), jnp.ones((1, 16), _f32))
    PADM = jnp.concatenate([jnp.eye(H, dtype=_f32),
                            jnp.zeros((H, 128 - H), _f32)], 1)
    return dict(H=H, FCH=FCH, W_e=W_e, b_e=b_e, W2bd=W2bd, W3bd=W3bd,
                A2M=A2M, A3M=A3M, EXPM=EXPM, PADM=PADM)


def kernel(h, e, edge_index, params):
    n = h.shape[0]
    src = edge_index[0]
    dst = edge_index[1]
    pad = E_PAD - src.shape[0]
    src_p = jnp.pad(src, (0, pad)).reshape(NW, NCHUNK, CH)
    dst_p = jnp.pad(dst, (0, pad)).reshape(NW, NCHUNK, CH)
    e_p = jnp.pad(e, ((0, pad), (0, 0)))

    zeros_acc = jnp.zeros((N_PAD, 128), _f32)
    gather_h = _make_gather2(n, 128, 128)
    scatter = _make_scatter(N_PAD, 128)

    hh = _mm(h, params['emb_h_W'], params['emb_h_b'], blk=1000)
    ee = _mm(e_p, params['emb_e_W'], params['emb_e_b'])

    for heads in params['layers']:
        lw = _prep_layer(heads)
        H = lw['H']
        Z = _mm(hh, lw['FCH'], jnp.zeros((128,), _f32), blk=1000)
        EP = _mm(ee, lw['W_e'], lw['b_e'])
        G1, G2 = gather_h(Z, Z, src_p, dst_p)
        LR, e_new = _edge_logits(EP, G1, G2, ee, lw['A2M'], lw['A3M'],
                                 lw['W2bd'], lw['W3bd'], H)
        M = _rowmax(LR)
        Wn, Wx = _edge_weights(G1, LR, M, lw['EXPM'], lw['PADM'], H)
        SN = scatter(Wn, dst_p, zeros_acc)
        SS = scatter(Wx, dst_p, zeros_acc)
        hh = _node_finalize(SN[0, :n], SN[1, :n], SS[0, :n], SS[1, :n], hh,
                            lw['EXPM'], H)
        ee = e_new

    HS, HD = gather_h(hh, hh, src_p, dst_p)
    mlp = params['mlp']
    W0 = mlp[0]['W']
    Wa, Wb, Wc = W0[:128], W0[128:256], W0[256:]
    W2p = jnp.zeros((mlp[2]['W'].shape[0], 128), _f32).at[:, :4].set(mlp[2]['W'])
    b2p = jnp.zeros((128,), _f32).at[:4].set(mlp[2]['b'])
    y = _readout(HS, HD, ee, Wa, Wb, Wc, mlp[0]['b'], mlp[1]['W'], mlp[1]['b'],
                 W2p, b2p)
    return y[:N_EDGES, :4]
